# baseline (speedup 1.0000x reference)
import jax
import jax.numpy as jnp
import numpy as np
from jax.experimental import pallas as pl
from jax.experimental.pallas import tpu as pltpu

GRID = 2048
DIRS8 = np.array(
    [[1, 0], [-1, 0], [0, 1], [0, -1], [1, 1], [1, -1], [-1, 1], [-1, -1]],
    dtype=np.float32,
)
UNIT8 = DIRS8 / np.linalg.norm(DIRS8, axis=1, keepdims=True)
DIRS8_I = DIRS8.astype(np.int32)
NEG_INF = float(np.float32(-np.inf))


def _dipole_kernel(*refs):
    r8_refs = refs[:8]
    m2x_ref, m2y_ref = refs[8:]
    m2x = jnp.zeros_like(r8_refs[0][...])
    m2y = jnp.zeros_like(m2x)
    for k in range(8):
        rk = r8_refs[k][...]
        m2x = m2x + rk * float(UNIT8[k, 0])
        m2y = m2y + rk * float(UNIT8[k, 1])
    m2x_ref[...] = m2x
    m2y_ref[...] = m2y


def _dv_kernel(beam_ref, *refs):
    r8_refs = refs[:8]
    (gxp_ref, gxm_ref, gyp_ref, gym_ref, ys_ref, xs_ref, orig_ref,
     dv_ref) = refs[8:]
    n = ys_ref.shape[0]

    r8 = [r[...] for r in r8_refs]
    mxx = jnp.zeros((n,), jnp.float32)
    myy = jnp.zeros((n,), jnp.float32)
    mxy = jnp.zeros((n,), jnp.float32)
    m2x = jnp.zeros((n,), jnp.float32)
    m2y = jnp.zeros((n,), jnp.float32)
    for k in range(8):
        ux = float(UNIT8[k, 0])
        uy = float(UNIT8[k, 1])
        mxx = mxx + r8[k] * (ux * ux)
        myy = myy + r8[k] * (uy * uy)
        mxy = mxy + r8[k] * (ux * uy)
        m2x = m2x + r8[k] * ux
        m2y = m2y + r8[k] * uy

    div = 0.5 * (gxp_ref[...] - gxm_ref[...]) + 0.5 * (gyp_ref[...] -
                                                       gym_ref[...])

    t = mxx + myy
    rad = jnp.sqrt((mxx - myy) ** 2 + 4.0 * mxy**2 + 1e-20)
    lam2 = 0.5 * (t - rad)
    c = jnp.maximum(lam2 / (t + 1e-09), 0.0)
    tr = jnp.maximum(t, 1e-09)
    s = jnp.maximum(div, 0.0) / (tr + 1e-09)

    b0 = beam_ref[0]
    b1 = beam_ref[1]
    bn = jax.lax.sqrt(b0 * b0 + b1 * b1) + 1e-09
    b0 = b0 / bn
    b1 = b1 / bn
    m_norm = jnp.sqrt(m2x * m2x + m2y * m2y) + 1e-09
    fwd = jnp.maximum(m2x * b0 + m2y * b1, 0.0) / m_norm

    rplus = jnp.zeros((n,), jnp.float32)
    rminus = jnp.zeros((n,), jnp.float32)
    for k in range(8):
        proj = float(UNIT8[k, 0]) * b0 + float(UNIT8[k, 1]) * b1
        rplus = rplus + r8[k] * (proj > 0).astype(jnp.float32)
        rminus = rminus + r8[k] * (proj < 0).astype(jnp.float32)
    bpv = rplus / (rplus + rminus + 1e-09)

    lex = s + 0.001 * bpv + 1e-06 * c + 1e-09 * fwd

    orig = orig_ref[...]
    maxlex = jnp.max(lex)
    pvo = jnp.min(jnp.where(lex == maxlex, orig, jnp.int32(n)))
    sel = (orig == pvo).astype(jnp.float32)
    ysf = ys_ref[...].astype(jnp.float32)
    xsf = xs_ref[...].astype(jnp.float32)
    pv_y = jnp.sum(sel * ysf)
    pv_x = jnp.sum(sel * xsf)

    dyy = ysf - pv_y
    dxx = xsf - pv_x
    r2 = dyy * dyy + dxx * dxx
    r_pix = jnp.sqrt(jnp.sum(r2) / n) + 1e-09
    disp = jnp.maximum((dyy * b0 + dxx * b1) / r_pix, 0.0)
    rnorm = jnp.sqrt(r2) + 1e-09
    hf = jnp.maximum((m2x * dyy + m2y * dxx) / rnorm, 0.0) / m_norm
    q = 1.0 - c / jnp.maximum(jnp.max(c), 1e-09)
    dv_ref[...] = c * disp * fwd * hf * q


def _nms_kernel(*refs):
    dv_ref = refs[0]
    nv_refs = refs[1:9]
    fm_refs = refs[9:17]
    out_ref = refs[17]
    dv = dv_ref[...]
    pooled = dv
    for k in range(8):
        nv = jnp.where(fm_refs[k][...] > 0, nv_refs[k][...],
                       jnp.float32(NEG_INF))
        pooled = jnp.maximum(pooled, nv)
    is_peak = (dv >= pooled) & (dv > 0)
    out_ref[...] = jnp.where(is_peak, dv, 0.0)


def _vspec():
    return pl.BlockSpec()


@jax.jit
def kernel(coords, feats, beam):
    n = coords.shape[0]
    y = coords[:, 0].astype(jnp.int32)
    x = coords[:, 1].astype(jnp.int32)
    ckey = y * GRID + x
    order = jnp.argsort(ckey)
    skey = ckey[order]
    ys = y[order]
    xs = x[order]
    occ_s = (feats[order, 0] > 0).astype(jnp.float32)

    pos8 = []
    found8 = []
    for k in range(8):
        dx = int(DIRS8_I[k, 0])
        dy = int(DIRS8_I[k, 1])
        ny = ys + dy
        nx = xs + dx
        valid = (ny >= 0) & (ny < GRID) & (nx >= 0) & (nx < GRID)
        nkey = ny * GRID + nx
        pos = jnp.clip(jnp.searchsorted(skey, nkey), 0, n - 1)
        found = valid & (skey[pos] == nkey)
        pos8.append(pos)
        found8.append(found)

    r8 = [jnp.where(found8[k], occ_s[pos8[k]], 0.0) for k in range(8)]

    m2x, m2y = pl.pallas_call(
        _dipole_kernel,
        out_shape=[jax.ShapeDtypeStruct((n,), jnp.float32)] * 2,
    )(*r8)

    gxp = jnp.where(found8[0], m2x[pos8[0]], 0.0)
    gxm = jnp.where(found8[1], m2x[pos8[1]], 0.0)
    gyp = jnp.where(found8[2], m2y[pos8[2]], 0.0)
    gym = jnp.where(found8[3], m2y[pos8[3]], 0.0)

    dv = pl.pallas_call(
        _dv_kernel,
        out_shape=jax.ShapeDtypeStruct((n,), jnp.float32),
        in_specs=[pl.BlockSpec(memory_space=pltpu.SMEM)] + [_vspec()] * 15,
        out_specs=_vspec(),
    )(beam, *r8, gxp, gxm, gyp, gym, ys, xs, order.astype(jnp.int32))

    nv8 = [dv[pos8[k]] for k in range(8)]
    fm8 = [found8[k].astype(jnp.float32) for k in range(8)]
    out_s = pl.pallas_call(
        _nms_kernel,
        out_shape=jax.ShapeDtypeStruct((n,), jnp.float32),
    )(dv, *nv8, *fm8)

    return jnp.zeros((n,), jnp.float32).at[order].set(out_s)
